# baseline (speedup 1.0000x reference)
import jax
import jax.numpy as jnp
from jax import lax
from jax.experimental import pallas as pl
from jax.experimental.pallas import tpu as pltpu
from jax.experimental.pallas import tpu_sc as plsc

_L = 16
_M = 9


def _collapse(p):
    W1, b1, W2, b2, W3, b3 = p
    A = (W3 @ W2 @ W1).T
    b = W3 @ (W2 @ b1 + b2) + b3
    return A.astype(jnp.float32), b.reshape(1, -1).astype(jnp.float32)


def _pad_rows(A, rows):
    return jnp.pad(A, ((0, rows - A.shape[0]), (0, 0)))


def _blockdiag9(A):
    k, f = A.shape
    out = jnp.zeros((_M * k, _M * f), jnp.float32)
    for j in range(_M):
        out = out.at[j * k:(j + 1) * k, j * f:(j + 1) * f].set(A)
    return out


def _tile9(b):
    return jnp.tile(b, (1, _M))


def _sc_gather(table, idx, n_pad, c_nodes, n_chunks):
    nodes_w = n_pad // 32
    mesh = plsc.VectorSubcoreMesh(
        core_axis_name="c", subcore_axis_name="s", num_cores=2, num_subcores=16
    )

    def body(table_hbm, idx_hbm, out_hbm, idx_v, rows_v, sem):
        wid = lax.axis_index("s") * 2 + lax.axis_index("c")
        for c in range(n_chunks):
            e0 = (wid * nodes_w + c * c_nodes) * _M
            pltpu.sync_copy(idx_hbm.at[pl.ds(e0, c_nodes * _M)], idx_v)
            pltpu.async_copy(table_hbm.at[idx_v], rows_v, sem).wait()
            pltpu.sync_copy(rows_v, out_hbm.at[pl.ds(e0, c_nodes * _M)])

    kfn = pl.kernel(
        body,
        out_type=jax.ShapeDtypeStruct((n_pad * _M, _L), jnp.float32),
        mesh=mesh,
        scratch_types=[
            pltpu.VMEM((c_nodes * _M,), jnp.int32),
            pltpu.VMEM((c_nodes * _M, _L), jnp.float32),
            pltpu.SemaphoreType.DMA,
        ],
        compiler_params=pltpu.CompilerParams(use_tc_tiling_on_sc=False),
    )
    return kfn(table, idx)


def _dot(a, b):
    return jnp.dot(a, b, preferred_element_type=jnp.float32)


def _relu(t):
    return jnp.maximum(t, 0.0)


def _node_body(*refs):
    (x_ref, cfe_ref,
     a11, b11, a21, b21, a31, b31, a41, b41,
     w4, w3, w1, e0, aend, bend, p_ref, cfeo_ref) = refs
    x = x_ref[...]
    f1 = _relu(_dot(x, a11[...]) + b11[...])
    f2 = _relu(_dot(f1, a21[...]) + b21[...])
    f3 = _relu(_dot(f2, a31[...]) + b31[...])
    f4 = _relu(_dot(f3, a41[...]) + b41[...])
    p_ref[...] = (_dot(f4, w4[...]) + _dot(f3, w3[...]) + _dot(f1, w1[...])
                  + _dot(x, e0[...]) + bend[...])
    cfeo_ref[...] = _relu(_dot(cfe_ref[...], aend[...]) + bend[...])


def _edge_body(*refs):
    (x_ref, g_ref, p_ref, cfeo_ref,
     w22bd, a22n9, c22t, w11bd, b11t, w21bd, b21t, a23, b23,
     e2, out_ref) = refs
    x = x_ref[...]
    g = g_ref[...]
    gates = _relu(_dot(g, w22bd[...]) + _dot(x, a22n9[...]) + c22t[...])
    f1g = _relu(_dot(g, w11bd[...]) + b11t[...])
    f2g = _relu(_dot(f1g, w21bd[...]) + b21t[...])
    f21 = _relu(_dot(f2g * gates, a23[...]) + b23[...])
    out1 = _relu(_dot(f21, e2[...]) + p_ref[...])
    out_ref[...] = jnp.concatenate([out1, cfeo_ref[...]], axis=1)


def kernel(color_point_fea, color_point_link, color_features_expand,
           fc11, fc21, fc31, fc41, fc22, fc23, fc_end):
    n, m = color_point_fea.shape
    assert m == _M

    A11, b11 = _collapse(fc11)
    A21, b21 = _collapse(fc21)
    A31, b31 = _collapse(fc31)
    A41, b41 = _collapse(fc41)
    A22, c22 = _collapse(fc22)
    A23, b23 = _collapse(fc23)
    Aend, bend = _collapse(fc_end)
    A11p = _pad_rows(A11, _L)
    A22p = _pad_rows(A22, _L)
    W22bd = _blockdiag9(A22p)
    A22n9 = jnp.tile(-A22p, (1, _M))
    C22t = _tile9(c22)
    W11bd = _blockdiag9(A11p)
    B11t = _tile9(b11)
    W21bd = _blockdiag9(A21)
    B21t = _tile9(b21)
    CCW = jnp.concatenate(
        [Aend[0:6], Aend[6:24], Aend[24:78], Aend[78:96],
         _pad_rows(Aend[96:102], _L)], axis=0)

    x_pad = jnp.pad(color_point_fea, ((0, 0), (0, _L - m)))
    n_pad = -(-n // 256) * 256
    e, e_pad = n * m, n_pad * m
    idx = jnp.concatenate([
        color_point_link.astype(jnp.int32),
        jnp.arange(e_pad - e, dtype=jnp.int32) % n,
    ])

    nodes_w = n_pad // 32
    c_nodes = next(k for k in range(nodes_w, 0, -1)
                   if nodes_w % k == 0 and k % 8 == 0
                   and k * _M * (4 * _L + 4) <= 450_000)
    G = _sc_gather(x_pad, idx, n_pad, c_nodes, nodes_w // c_nodes)
    G = G.reshape(n_pad, _M * _L)

    R = 4000
    assert n % R == 0
    bspec = lambda shape: pl.BlockSpec(shape, lambda i: (i, 0))
    wspec = lambda w: pl.BlockSpec(w.shape, lambda i: (0, 0))

    node_w = (A11p, b11, A21, b21, A31, b31, A41, b41,
              CCW[0:6], CCW[6:24], CCW[78:96], CCW[96:112], Aend, bend)
    P, cfeo = pl.pallas_call(
        _node_body,
        grid=(n // R,),
        in_specs=[bspec((R, _L)), bspec((R, 102))] + [wspec(w) for w in node_w],
        out_specs=[bspec((R, 51)), bspec((R, 51))],
        out_shape=[jax.ShapeDtypeStruct((n, 51), jnp.float32),
                   jax.ShapeDtypeStruct((n, 51), jnp.float32)],
    )(x_pad, color_features_expand, *node_w)

    edge_w = (W22bd, A22n9, C22t, W11bd, B11t, W21bd, B21t, A23, b23,
              CCW[24:78])
    out = pl.pallas_call(
        _edge_body,
        grid=(n // R,),
        in_specs=[bspec((R, _L)), bspec((R, _M * _L)), bspec((R, 51)),
                  bspec((R, 51))] + [wspec(w) for w in edge_w],
        out_specs=bspec((R, 102)),
        out_shape=jax.ShapeDtypeStruct((n, 102), jnp.float32),
    )(x_pad, G, P, cfeo, *edge_w)
    return out


# speedup vs baseline: 1.0974x; 1.0974x over previous
import jax
import jax.numpy as jnp
from jax import lax
from jax.experimental import pallas as pl
from jax.experimental.pallas import tpu as pltpu
from jax.experimental.pallas import tpu_sc as plsc

_L = 16
_M = 9


def _collapse(p):
    W1, b1, W2, b2, W3, b3 = p
    A = (W3 @ W2 @ W1).T
    b = W3 @ (W2 @ b1 + b2) + b3
    return A.astype(jnp.float32), b.reshape(1, -1).astype(jnp.float32)


def _pad_rows(A, rows):
    return jnp.pad(A, ((0, rows - A.shape[0]), (0, 0)))


def _blockdiag9(A):
    k, f = A.shape
    out = jnp.zeros((_M * k, _M * f), jnp.float32)
    for j in range(_M):
        out = out.at[j * k:(j + 1) * k, j * f:(j + 1) * f].set(A)
    return out


def _tile9(b):
    return jnp.tile(b, (1, _M))


def _sc_gather(table, idx, n_pad, c_nodes, n_chunks):
    nodes_w = n_pad // 32
    mesh = plsc.VectorSubcoreMesh(
        core_axis_name="c", subcore_axis_name="s", num_cores=2, num_subcores=16
    )

    def body(table_hbm, idx_hbm, out_hbm, idx_v, rows_v, sem):
        wid = lax.axis_index("s") * 2 + lax.axis_index("c")
        for c in range(n_chunks):
            e0 = (wid * nodes_w + c * c_nodes) * _M
            pltpu.sync_copy(idx_hbm.at[pl.ds(e0, c_nodes * _M)], idx_v)
            pltpu.async_copy(table_hbm.at[idx_v], rows_v, sem).wait()
            pltpu.sync_copy(rows_v, out_hbm.at[pl.ds(e0, c_nodes * _M)])

    kfn = pl.kernel(
        body,
        out_type=jax.ShapeDtypeStruct((n_pad * _M, _L), jnp.float32),
        mesh=mesh,
        scratch_types=[
            pltpu.VMEM((c_nodes * _M,), jnp.int32),
            pltpu.VMEM((c_nodes * _M, _L), jnp.float32),
            pltpu.SemaphoreType.DMA,
        ],
        compiler_params=pltpu.CompilerParams(use_tc_tiling_on_sc=False),
    )
    return kfn(table, idx)


def _dense_body(*refs):
    (x_ref, g_ref, cfe_ref,
     a11, b11, a21, b21, a31, b31, a41, b41,
     w22bd, a22n9, c22t, w11bd, b11t, w21bd, b21t, a23, b23,
     ccw, aend, bend, out_ref) = refs

    def dot(a, b):
        return jnp.dot(a, b, preferred_element_type=jnp.float32)

    relu = lambda t: jnp.maximum(t, 0.0)
    bf = lambda t: t.astype(jnp.bfloat16)

    x = x_ref[...]
    g = bf(g_ref[...])
    f1 = relu(dot(x, a11[...]) + b11[...])
    f2 = relu(dot(f1, a21[...]) + b21[...])
    f3 = relu(dot(f2, a31[...]) + b31[...])
    f4 = relu(dot(f3, a41[...]) + b41[...])

    gates = relu(dot(g, w22bd[...]) + dot(x, a22n9[...]) + c22t[...])
    f1g = relu(dot(g, w11bd[...]) + b11t[...])
    f2g = relu(dot(bf(f1g), w21bd[...]) + b21t[...])
    f21 = relu(dot(bf(f2g * gates), a23[...]) + b23[...])

    cc = jnp.concatenate([f4, f3, f21, f1, x], axis=1)
    out1 = relu(dot(cc, ccw[...]) + bend[...])
    cfeo = relu(dot(bf(cfe_ref[...]), aend[...]) + bend[...])
    out_ref[...] = jnp.concatenate([out1, cfeo], axis=1)


def kernel(color_point_fea, color_point_link, color_features_expand,
           fc11, fc21, fc31, fc41, fc22, fc23, fc_end):
    n, m = color_point_fea.shape
    assert m == _M

    A11, b11 = _collapse(fc11)
    A21, b21 = _collapse(fc21)
    A31, b31 = _collapse(fc31)
    A41, b41 = _collapse(fc41)
    A22, c22 = _collapse(fc22)
    A23, b23 = _collapse(fc23)
    Aend, bend = _collapse(fc_end)
    A11p = _pad_rows(A11, _L)
    A22p = _pad_rows(A22, _L)
    W22bd = _blockdiag9(A22p)
    A22n9 = jnp.tile(-A22p, (1, _M))
    C22t = _tile9(c22)
    W11bd = _blockdiag9(A11p)
    B11t = _tile9(b11)
    W21bd = _blockdiag9(A21)
    B21t = _tile9(b21)
    CCW = jnp.concatenate(
        [Aend[0:6], Aend[6:24], Aend[24:78], Aend[78:96],
         _pad_rows(Aend[96:102], _L)], axis=0)

    x_pad = jnp.pad(color_point_fea, ((0, 0), (0, _L - m)))
    n_pad = -(-n // 256) * 256
    e, e_pad = n * m, n_pad * m
    idx = jnp.concatenate([
        color_point_link.astype(jnp.int32),
        jnp.arange(e_pad - e, dtype=jnp.int32) % n,
    ])

    nodes_w = n_pad // 32
    c_nodes = next(k for k in range(nodes_w, 0, -1)
                   if nodes_w % k == 0 and k % 8 == 0
                   and k * _M * (4 * _L + 4) <= 450_000)
    G = _sc_gather(x_pad, idx, n_pad, c_nodes, nodes_w // c_nodes)
    G = G.reshape(n_pad, _M * _L)

    R = 4000
    assert n % R == 0
    bf16 = jnp.bfloat16
    weights = (A11p, b11, A21, b21, A31, b31, A41, b41,
               W22bd.astype(bf16), A22n9, C22t, W11bd.astype(bf16), B11t,
               W21bd.astype(bf16), B21t, A23.astype(bf16), b23,
               CCW, Aend.astype(bf16), bend)
    bspec = lambda shape: pl.BlockSpec(shape, lambda i: (i, 0))
    wspec = lambda w: pl.BlockSpec(w.shape, lambda i: (0, 0))
    out = pl.pallas_call(
        _dense_body,
        grid=(n // R,),
        in_specs=[bspec((R, _L)), bspec((R, _M * _L)), bspec((R, 102))]
        + [wspec(w) for w in weights],
        out_specs=bspec((R, 102)),
        out_shape=jax.ShapeDtypeStruct((n, 102), jnp.float32),
    )(x_pad, G, color_features_expand, *weights)
    return out
